# Initial kernel scaffold; baseline (speedup 1.0000x reference)
import jax
import jax.numpy as jnp
from jax.experimental import pallas as pl


def kernel(z_e, embedding):
    raise NotImplementedError("write your pallas kernel here")


# speedup vs baseline: 1.5172x; 1.5172x over previous
import functools

import jax
import jax.numpy as jnp
from jax import lax
from jax.experimental import pallas as pl
from jax.experimental.pallas import tpu as pltpu

K = 8192
D = 256
T_BLK = 256
COMMIT = 0.5


def _vq_kernel(z_ref, e_ref, zq_ref, idx_ref, counts_ref, loss_ref, perp_ref,
               en_ref, *, nb, nt, n_total):
    b = pl.program_id(0)
    t = pl.program_id(1)
    first = jnp.logical_and(b == 0, t == 0)
    last = jnp.logical_and(b == nb - 1, t == nt - 1)

    z = z_ref[0]
    e = e_ref[...]

    @pl.when(first)
    def _init():
        en_ref[...] = jnp.sum(e * e, axis=1, keepdims=True)
        counts_ref[...] = jnp.zeros_like(counts_ref)
        loss_ref[0, 0] = 0.0

    s = lax.dot_general(e.astype(jnp.bfloat16), z.astype(jnp.bfloat16),
                        (((1,), (0,)), ((), ())),
                        preferred_element_type=jnp.float32)
    zn = jnp.sum(z * z, axis=0, keepdims=True)
    d = jnp.maximum(en_ref[...] + zn - 2.0 * s, 0.0)

    iota_k = lax.broadcasted_iota(jnp.int32, (K, 1), 0)
    idx = jnp.argmin(d, axis=0).astype(jnp.int32)
    idx_ref[0, 0, :] = idx

    oh = (iota_k == idx[None, :]).astype(jnp.float32)
    zq = lax.dot_general(e.astype(jnp.bfloat16), oh.astype(jnp.bfloat16),
                         (((0,), (0,)), ((), ())),
                         preferred_element_type=jnp.float32)
    zq_ref[0] = z + (zq - z)

    counts_ref[...] += jnp.sum(oh, axis=1, keepdims=True)
    diff = zq - z
    loss_ref[0, 0] += jnp.sum(diff * diff)

    @pl.when(last)
    def _fin():
        loss_ref[0, 0] = loss_ref[0, 0] * ((1.0 + COMMIT) / n_total)
        c = counts_ref[...]
        e_mean = c / (jnp.sum(c) + 1e-08)
        ent = -jnp.sum(e_mean * jnp.log(e_mean + 1e-08))
        perp_ref[0, 0] = jnp.exp(ent)


def kernel(z_e, embedding):
    B, Dd, T = z_e.shape
    nb, nt = B, T // T_BLK
    n_total = B * T * Dd

    grid = (nb, nt)
    zq, idx3, counts, loss, perp = pl.pallas_call(
        functools.partial(_vq_kernel, nb=nb, nt=nt, n_total=n_total),
        grid=grid,
        in_specs=[
            pl.BlockSpec((1, Dd, T_BLK), lambda b, t: (b, 0, t)),
            pl.BlockSpec((K, Dd), lambda b, t: (0, 0)),
        ],
        out_specs=[
            pl.BlockSpec((1, Dd, T_BLK), lambda b, t: (b, 0, t)),
            pl.BlockSpec((1, 1, T_BLK), lambda b, t: (b * nt + t, 0, 0)),
            pl.BlockSpec((K, 1), lambda b, t: (0, 0)),
            pl.BlockSpec(memory_space=pltpu.SMEM),
            pl.BlockSpec(memory_space=pltpu.SMEM),
        ],
        out_shape=[
            jax.ShapeDtypeStruct((B, Dd, T), jnp.float32),
            jax.ShapeDtypeStruct((nb * nt, 1, T_BLK), jnp.int32),
            jax.ShapeDtypeStruct((K, 1), jnp.float32),
            jax.ShapeDtypeStruct((1, 1), jnp.float32),
            jax.ShapeDtypeStruct((1, 1), jnp.float32),
        ],
        scratch_shapes=[pltpu.VMEM((K, 1), jnp.float32)],
    )(z_e, embedding)

    min_idx = idx3.reshape(B, T)
    return (zq, loss[0, 0], perp[0, 0], min_idx)
